# baseline (speedup 1.0000x reference)
import functools

import jax
import jax.numpy as jnp
from jax import lax
from jax.experimental import pallas as pl
from jax.experimental.pallas import tpu as pltpu
from jax.experimental.pallas import tpu_sc as plsc

_B = 4
_T = 4096
_H = 1024
_P = 512
_O = 768
_CW = 128
_NS = 16
_G = 8
_CG = _H // _G
_CHUNK = 32

_TPH = _T // 2
_NCH = _TPH // _CHUNK
_PH = _P // 2
_ZR = 128


def _sc_pool_pair(h, pid3, zsum, ones, base):
    mesh = plsc.VectorSubcoreMesh(core_axis_name="c", subcore_axis_name="s")

    @functools.partial(
        pl.kernel,
        out_type=[
            jax.ShapeDtypeStruct((2, _G, _P, _CG), jnp.float32),
            jax.ShapeDtypeStruct((2, _P, _CW), jnp.float32),
        ],
        mesh=mesh,
        scratch_types=[
            pltpu.VMEM((_ZR, _CG), jnp.float32),
            pltpu.VMEM((_CHUNK, _CG), jnp.float32),
            pltpu.VMEM((_CHUNK, _CG), jnp.float32),
            pltpu.VMEM((_NCH, _CHUNK), jnp.int32),
            pltpu.VMEM_SHARED((_G, _P, _CG), jnp.float32),
            pltpu.VMEM_SHARED((_P, _CW), jnp.float32),
            pltpu.SemaphoreType.DMA,
            pltpu.SemaphoreType.DMA,
            pltpu.SemaphoreType.DMA,
            pltpu.SemaphoreType.DMA,
        ],
    )
    def k(h_hbm, pid_hbm, zsum_hbm, ones_hbm, sums_hbm, cnts_hbm,
          zero_v, chunk0_v, chunk1_v, idx_v, acc, cacc,
          gsem0, gsem1, ssem0, ssem1):
        c = lax.axis_index("c")
        s = lax.axis_index("s")
        g = s % _G
        th = s // _G
        col0 = g * _CG
        tok0 = th * _TPH
        b = base + c
        bufs = (chunk0_v, chunk1_v)
        gsems = (gsem0, gsem1)
        ssems = (ssem0, ssem1)

        pltpu.sync_copy(zsum_hbm, zero_v)
        for z in range(_PH // _ZR):
            pltpu.sync_copy(
                zero_v, acc.at[s // 2, pl.ds((s % 2) * _PH + z * _ZR, _ZR)])

        @pl.when(s // 2 == 0)
        def _():
            for z in range(_PH // _ZR):
                pltpu.sync_copy(
                    zero_v, cacc.at[pl.ds((s % 2) * _PH + z * _ZR, _ZR)])
        plsc.subcore_barrier()

        def gather(j, k2):
            return pltpu.async_copy(
                h_hbm.at[b, pl.ds(tok0 + j * _CHUNK, _CHUNK),
                         pl.ds(col0, _CG)],
                bufs[k2], gsems[k2])

        pltpu.sync_copy(pid_hbm.at[b, pl.ds(th * _NCH, _NCH)], idx_v)
        gd = [gather(0, 0), None]
        sd = [None, None]
        for j in range(_NCH):
            k2 = j % 2
            gd[k2].wait()
            sd[k2] = pltpu.async_copy(bufs[k2],
                                      acc.at[g].at[idx_v.at[j]],
                                      ssems[k2], add=True)
            if j + 1 < _NCH:
                o = 1 - k2
                if sd[o] is not None:
                    sd[o].wait()
                gd[o] = gather(j + 1, o)
        sd[0].wait()
        sd[1].wait()

        @pl.when(g == 0)
        def _():
            pltpu.sync_copy(ones_hbm, chunk0_v)
            for j in range(_NCH):
                pltpu.sync_copy(chunk0_v, cacc.at[idx_v.at[j]], add=True)
        plsc.subcore_barrier()

        pltpu.sync_copy(acc.at[s // 2, pl.ds((s % 2) * _PH, _PH)],
                        sums_hbm.at[c, s // 2, pl.ds((s % 2) * _PH, _PH)])

        @pl.when(s // 2 == 0)
        def _():
            pltpu.sync_copy(cacc.at[pl.ds((s % 2) * _PH, _PH)],
                            cnts_hbm.at[c, pl.ds((s % 2) * _PH, _PH)])

    return k(h, pid3, zsum, ones)


def _tc_project_body(sums_ref, cnts_ref, w_ref, b_ref, out_ref, acc_ref):
    k = pl.program_id(1)
    nk = pl.num_programs(1)

    @pl.when(k == 0)
    def _():
        acc_ref[...] = jnp.zeros_like(acc_ref)

    acc_ref[...] += jnp.dot(sums_ref[0, 0], w_ref[0],
                            preferred_element_type=jnp.float32)

    @pl.when(k == nk - 1)
    def _():
        cnt = cnts_ref[0, :, 0:1]
        inv = 1.0 / jnp.maximum(cnt, 1.0)
        out_ref[0] = acc_ref[...] * inv + b_ref[...]


def _tc_project(sums, cnts, w3, b2):
    return pl.pallas_call(
        _tc_project_body,
        grid=(2, _G),
        in_specs=[
            pl.BlockSpec((1, 1, _P, _CG), lambda b, k: (b, k, 0, 0)),
            pl.BlockSpec((1, _P, _CW), lambda b, k: (b, 0, 0)),
            pl.BlockSpec((1, _CG, _O), lambda b, k: (k, 0, 0)),
            pl.BlockSpec((1, _O), lambda b, k: (0, 0)),
        ],
        out_specs=pl.BlockSpec((1, _P, _O), lambda b, k: (b, 0, 0)),
        out_shape=jax.ShapeDtypeStruct((2, _P, _O), jnp.float32),
        scratch_shapes=[pltpu.VMEM((_P, _O), jnp.float32)],
    )(sums, cnts, w3, b2)


def kernel(byte_hiddens, patch_ids, W_proj, b_proj):
    pid3 = patch_ids.astype(jnp.int32).reshape(_B, _T // _CHUNK, _CHUNK)
    zsum = jnp.zeros((_ZR, _CG), jnp.float32)
    ones = jnp.zeros((_CHUNK, _CW), jnp.float32).at[:, 0].set(1.0)
    w3 = W_proj.reshape(_G, _CG, _O)
    b2 = b_proj.reshape(1, _O)
    sums0, cnts0 = _sc_pool_pair(byte_hiddens, pid3, zsum, ones, 0)
    sums1, cnts1 = _sc_pool_pair(byte_hiddens, pid3, zsum, ones, 2)
    out0 = _tc_project(sums0, cnts0, w3, b2)
    out1 = _tc_project(sums1, cnts1, w3, b2)
    return jnp.concatenate([out0, out1], axis=0)


# speedup vs baseline: 1.1984x; 1.1984x over previous
import functools

import jax
import jax.numpy as jnp
from jax import lax
from jax.experimental import pallas as pl
from jax.experimental.pallas import tpu as pltpu
from jax.experimental.pallas import tpu_sc as plsc

_B = 4
_T = 4096
_H = 1024
_P = 512
_O = 768
_NS = 16
_G = 8
_CG = _H // _G
_CHUNK = 16

_TPW = _T // _NS
_NCH = _TPW // _CHUNK
_PW = _P // _NS
_TK = _T // _G


def _sc_pool(h4, pid3, zsum):
    mesh = plsc.VectorSubcoreMesh(core_axis_name="c", subcore_axis_name="s")

    @functools.partial(
        pl.kernel,
        out_type=jax.ShapeDtypeStruct((_B, _P, _G, _CG), jnp.float32),
        mesh=mesh,
        scratch_types=[
            pltpu.VMEM((_CHUNK, _G, _CG), jnp.float32),
            pltpu.VMEM((_CHUNK, _G, _CG), jnp.float32),
            pltpu.VMEM((_NCH, _CHUNK), jnp.int32),
            pltpu.VMEM_SHARED((_P, _G, _CG), jnp.float32),
            pltpu.VMEM_SHARED((_P, _G, _CG), jnp.float32),
            pltpu.SemaphoreType.DMA,
            pltpu.SemaphoreType.DMA,
            pltpu.SemaphoreType.DMA,
            pltpu.SemaphoreType.DMA,
        ],
    )
    def k(h_hbm, pid_hbm, zsum_hbm, sums_hbm,
          chunk0_v, chunk1_v, idx_v, acc0, acc1,
          gsem0, gsem1, ssem0, ssem1):
        c = lax.axis_index("c")
        s = lax.axis_index("s")
        tok0 = s * _TPW
        bufs = (chunk0_v, chunk1_v)
        gsems = (gsem0, gsem1)
        ssems = (ssem0, ssem1)

        pltpu.sync_copy(zsum_hbm, chunk0_v)
        for acc in (acc0, acc1):
            for z in range(_PW // _CHUNK):
                pltpu.sync_copy(
                    chunk0_v, acc.at[pl.ds(s * _PW + z * _CHUNK, _CHUNK)])
        plsc.subcore_barrier()

        def gather(b, j, k2):
            return pltpu.async_copy(
                h_hbm.at[b, pl.ds(tok0 + j * _CHUNK, _CHUNK)],
                bufs[k2], gsems[k2])

        for bi, acc in enumerate((acc0, acc1)):
            b = c * 2 + bi
            pltpu.sync_copy(
                pid_hbm.at[b, pl.ds(s * _NCH, _NCH)], idx_v)
            gd = [gather(b, 0, 0), None]
            sd = [None, None]
            for j in range(_NCH):
                k2 = j % 2
                gd[k2].wait()
                sd[k2] = pltpu.async_copy(bufs[k2],
                                          acc.at[idx_v.at[j]],
                                          ssems[k2], add=True)
                if j + 1 < _NCH:
                    o = 1 - k2
                    if sd[o] is not None:
                        sd[o].wait()
                    gd[o] = gather(b, j + 1, o)
            sd[0].wait()
            sd[1].wait()
        plsc.subcore_barrier()

        for bi, acc in enumerate((acc0, acc1)):
            b = c * 2 + bi
            pltpu.sync_copy(acc.at[pl.ds(s * _PW, _PW)],
                            sums_hbm.at[b, pl.ds(s * _PW, _PW)])

    return k(h4, pid3, zsum)


def _tc_project_body(sums_ref, pid_ref, w_ref, b_ref, out_ref,
                     acc_ref, cnt_ref):
    k = pl.program_id(1)
    nk = pl.num_programs(1)

    @pl.when(k == 0)
    def _():
        acc_ref[...] = jnp.zeros_like(acc_ref)
        cnt_ref[...] = jnp.zeros_like(cnt_ref)

    acc_ref[...] += jnp.dot(sums_ref[0], w_ref[...],
                            preferred_element_type=jnp.float32)
    ids = pid_ref[0, k]
    patches = lax.broadcasted_iota(jnp.int32, (_P, _TK), 0)
    m = (patches == ids[None, :]).astype(jnp.float32)
    cnt_ref[...] += jnp.sum(m, axis=1, keepdims=True)

    @pl.when(k == nk - 1)
    def _():
        inv = 1.0 / jnp.maximum(cnt_ref[...], 1.0)
        out_ref[0] = acc_ref[...] * inv + b_ref[...]


def _tc_project(sums, pid, w, b2):
    return pl.pallas_call(
        _tc_project_body,
        grid=(_B, _G),
        in_specs=[
            pl.BlockSpec((1, _P, _CG), lambda b, k: (b, 0, k)),
            pl.BlockSpec((1, _G, _TK), lambda b, k: (b, 0, 0)),
            pl.BlockSpec((_CG, _O), lambda b, k: (k, 0)),
            pl.BlockSpec((1, _O), lambda b, k: (0, 0)),
        ],
        out_specs=pl.BlockSpec((1, _P, _O), lambda b, k: (b, 0, 0)),
        out_shape=jax.ShapeDtypeStruct((_B, _P, _O), jnp.float32),
        scratch_shapes=[pltpu.VMEM((_P, _O), jnp.float32),
                        pltpu.VMEM((_P, 1), jnp.float32)],
    )(sums, pid, w, b2)


def kernel(byte_hiddens, patch_ids, W_proj, b_proj):
    h4 = byte_hiddens.reshape(_B, _T, _G, _CG)
    pid = patch_ids.astype(jnp.int32)
    pid3 = pid.reshape(_B, _T // _CHUNK, _CHUNK)
    zsum = jnp.zeros((_CHUNK, _G, _CG), jnp.float32)
    sums = _sc_pool(h4, pid3, zsum)
    return _tc_project(sums.reshape(_B, _P, _H), pid.reshape(_B, _G, _TK),
                       W_proj, b_proj.reshape(1, _O))


# speedup vs baseline: 1.7911x; 1.4946x over previous
import functools

import jax
import jax.numpy as jnp
from jax import lax
from jax.experimental import pallas as pl
from jax.experimental.pallas import tpu as pltpu
from jax.experimental.pallas import tpu_sc as plsc

_B = 4
_T = 4096
_H = 1024
_P = 512
_O = 768
_NS = 16
_G = 8
_CG = _H // _G
_CHUNK = 16

_TPW = _T // _NS
_NCH = _TPW // _CHUNK
_PW = _P // _NS
_TK = _T // _G


def _sc_pool(h, pid3, zsum):
    mesh = plsc.VectorSubcoreMesh(core_axis_name="c", subcore_axis_name="s")

    @functools.partial(
        pl.kernel,
        out_type=jax.ShapeDtypeStruct((_B, _G, _P, _CG), jnp.float32),
        mesh=mesh,
        scratch_types=[
            pltpu.VMEM((_CHUNK, _G, _CG), jnp.float32),
            pltpu.VMEM((_CHUNK, _G, _CG), jnp.float32),
            pltpu.VMEM((_NCH, _CHUNK), jnp.int32),
            pltpu.VMEM_SHARED((_P, _G, _CG), jnp.float32),
            pltpu.VMEM_SHARED((_P, _G, _CG), jnp.float32),
            pltpu.SemaphoreType.DMA,
            pltpu.SemaphoreType.DMA,
            pltpu.SemaphoreType.DMA,
            pltpu.SemaphoreType.DMA,
        ],
    )
    def k(h_hbm, pid_hbm, zsum_hbm, sums_hbm,
          chunk0_v, chunk1_v, idx_v, acc0, acc1,
          gsem0, gsem1, ssem0, ssem1):
        c = lax.axis_index("c")
        s = lax.axis_index("s")
        tok0 = s * _TPW
        bufs = (chunk0_v, chunk1_v)
        gsems = (gsem0, gsem1)
        ssems = (ssem0, ssem1)

        pltpu.sync_copy(zsum_hbm, chunk0_v)
        for acc in (acc0, acc1):
            for z in range(_PW // _CHUNK):
                pltpu.sync_copy(
                    chunk0_v, acc.at[pl.ds(s * _PW + z * _CHUNK, _CHUNK)])
        plsc.subcore_barrier()

        def gather(b, j, k2):
            t0 = tok0 + j * _CHUNK
            for g in range(_G):
                pltpu.async_copy(
                    h_hbm.at[b, pl.ds(t0, _CHUNK), pl.ds(g * _CG, _CG)],
                    bufs[k2].at[:, g], gsems[k2])

        def drain(sem, buf):
            pltpu.make_async_copy(zsum_hbm, buf, sem).wait()

        for bi, acc in enumerate((acc0, acc1)):
            b = c * 2 + bi
            pltpu.sync_copy(
                pid_hbm.at[b, pl.ds(s * _NCH, _NCH)], idx_v)
            for b2 in range(2):
                gather(b, b2, b2)

            @pl.loop(0, _NCH, step=2)
            def _(jg):
                for b2 in range(2):
                    j = jg + b2
                    drain(gsems[b2], bufs[b2])
                    pltpu.async_copy(bufs[b2], acc.at[idx_v.at[j]],
                                     ssems[b2], add=True)

                    @pl.when(j + 2 < _NCH)
                    def _():
                        drain(ssems[b2], bufs[b2])
                        gather(b, j + 2, b2)

            for b2 in range(2):
                drain(ssems[b2], bufs[b2])
        plsc.subcore_barrier()

        for bi, acc in enumerate((acc0, acc1)):
            b = c * 2 + bi
            for g in range(_G):
                pltpu.sync_copy(
                    acc.at[pl.ds(s * _PW, _PW), g],
                    sums_hbm.at[b, g, pl.ds(s * _PW, _PW)])

    return k(h, pid3, zsum)


def _tc_project_body(sums_ref, pid_ref, w_ref, b_ref, out_ref,
                     acc_ref, cnt_ref):
    k = pl.program_id(1)
    nk = pl.num_programs(1)

    @pl.when(k == 0)
    def _():
        acc_ref[...] = jnp.zeros_like(acc_ref)
        cnt_ref[...] = jnp.zeros_like(cnt_ref)

    acc_ref[...] += jnp.dot(sums_ref[0, 0], w_ref[0],
                            preferred_element_type=jnp.float32)
    ids = pid_ref[0, k]
    patches = lax.broadcasted_iota(jnp.int32, (_P, _TK), 0)
    m = (patches == ids[None, :]).astype(jnp.float32)
    cnt_ref[...] += jnp.sum(m, axis=1, keepdims=True)

    @pl.when(k == nk - 1)
    def _():
        inv = 1.0 / jnp.maximum(cnt_ref[...], 1.0)
        out_ref[0] = acc_ref[...] * inv + b_ref[...]


def _tc_project(sums, pid, w3, b2):
    return pl.pallas_call(
        _tc_project_body,
        grid=(_B, _G),
        in_specs=[
            pl.BlockSpec((1, 1, _P, _CG), lambda b, k: (b, k, 0, 0)),
            pl.BlockSpec((1, _G, _TK), lambda b, k: (b, 0, 0)),
            pl.BlockSpec((1, _CG, _O), lambda b, k: (k, 0, 0)),
            pl.BlockSpec((1, _O), lambda b, k: (0, 0)),
        ],
        out_specs=pl.BlockSpec((1, _P, _O), lambda b, k: (b, 0, 0)),
        out_shape=jax.ShapeDtypeStruct((_B, _P, _O), jnp.float32),
        scratch_shapes=[pltpu.VMEM((_P, _O), jnp.float32),
                        pltpu.VMEM((_P, 1), jnp.float32)],
    )(sums, pid, w3, b2)


def kernel(byte_hiddens, patch_ids, W_proj, b_proj):
    pid = patch_ids.astype(jnp.int32)
    pid3 = pid.reshape(_B, _T // _CHUNK, _CHUNK)
    zsum = jnp.zeros((_CHUNK, _G, _CG), jnp.float32)
    sums = _sc_pool(byte_hiddens, pid3, zsum)
    return _tc_project(sums, pid.reshape(_B, _G, _TK),
                       W_proj.reshape(_G, _CG, _O), b_proj.reshape(1, _O))


# speedup vs baseline: 1.7931x; 1.0011x over previous
import functools

import jax
import jax.numpy as jnp
from jax import lax
from jax.experimental import pallas as pl
from jax.experimental.pallas import tpu as pltpu
from jax.experimental.pallas import tpu_sc as plsc

_B = 4
_T = 4096
_H = 1024
_P = 512
_O = 768
_NS = 16
_G = 8
_CG = _H // _G
_CHUNK = 16

_TPW = _T // _NS
_NCH = _TPW // _CHUNK
_PW = _P // _NS
_TK = _T // _G


def _sc_pool(h, pid3, zsum):
    mesh = plsc.VectorSubcoreMesh(core_axis_name="c", subcore_axis_name="s")

    @functools.partial(
        pl.kernel,
        out_type=jax.ShapeDtypeStruct((_B, _G, _P, _CG), jnp.float32),
        mesh=mesh,
        scratch_types=[
            pltpu.VMEM((_CHUNK, _G, _CG), jnp.float32),
            pltpu.VMEM((_CHUNK, _G, _CG), jnp.float32),
            pltpu.VMEM((_NCH, _CHUNK), jnp.int32),
            pltpu.VMEM_SHARED((_P, _G, _CG), jnp.float32),
            pltpu.VMEM_SHARED((_P, _G, _CG), jnp.float32),
            pltpu.SemaphoreType.DMA,
            pltpu.SemaphoreType.DMA,
            pltpu.SemaphoreType.DMA,
            pltpu.SemaphoreType.DMA,
        ],
    )
    def k(h_hbm, pid_hbm, zsum_hbm, sums_hbm,
          chunk0_v, chunk1_v, idx_v, acc0, acc1,
          gsem0, gsem1, ssem0, ssem1):
        c = lax.axis_index("c")
        s = lax.axis_index("s")
        tok0 = s * _TPW
        bufs = (chunk0_v, chunk1_v)
        gsems = (gsem0, gsem1)
        ssems = (ssem0, ssem1)

        pltpu.sync_copy(zsum_hbm, chunk0_v)
        for acc in (acc0, acc1):
            for z in range(_PW // _CHUNK):
                pltpu.sync_copy(
                    chunk0_v, acc.at[pl.ds(s * _PW + z * _CHUNK, _CHUNK)])
        plsc.subcore_barrier()

        def gather(b, j, k2):
            t0 = tok0 + j * _CHUNK
            for g in range(_G):
                pltpu.async_copy(
                    h_hbm.at[b, pl.ds(t0, _CHUNK), pl.ds(g * _CG, _CG)],
                    bufs[k2].at[:, g], gsems[k2])

        def drain(sem, buf):
            pltpu.make_async_copy(zsum_hbm, buf, sem).wait()

        for bi, acc in enumerate((acc0, acc1)):
            b = c * 2 + bi
            pltpu.sync_copy(
                pid_hbm.at[b, pl.ds(s * _NCH, _NCH)], idx_v)
            for b2 in range(2):
                gather(b, b2, b2)

            @pl.loop(0, _NCH, step=2)
            def _(jg):
                for b2 in range(2):
                    j = jg + b2
                    drain(gsems[b2], bufs[b2])
                    pltpu.async_copy(bufs[b2], acc.at[idx_v.at[j]],
                                     ssems[b2], add=True)

                    @pl.when(j + 2 < _NCH)
                    def _():
                        drain(ssems[b2], bufs[b2])
                        gather(b, j + 2, b2)

            for b2 in range(2):
                drain(ssems[b2], bufs[b2])
        plsc.subcore_barrier()

        for bi, acc in enumerate((acc0, acc1)):
            b = c * 2 + bi
            for g in range(_G):
                pltpu.sync_copy(
                    acc.at[pl.ds(s * _PW, _PW), g],
                    sums_hbm.at[b, g, pl.ds(s * _PW, _PW)])

    return k(h, pid3, zsum)


def _tc_inv_body(pid_ref, inv_ref):
    cnt = jnp.zeros((_P, 1), jnp.float32)
    patches = lax.broadcasted_iota(jnp.int32, (_P, _TK), 0)
    for r in range(_G):
        ids = pid_ref[0, r]
        m = (patches == ids[None, :]).astype(jnp.float32)
        cnt = cnt + jnp.sum(m, axis=1, keepdims=True)
    inv_ref[0] = 1.0 / jnp.maximum(cnt, 1.0)


def _tc_inv(pid):
    return pl.pallas_call(
        _tc_inv_body,
        grid=(_B,),
        in_specs=[pl.BlockSpec((1, _G, _TK), lambda b: (b, 0, 0))],
        out_specs=pl.BlockSpec((1, _P, 1), lambda b: (b, 0, 0)),
        out_shape=jax.ShapeDtypeStruct((_B, _P, 1), jnp.float32),
    )(pid)


def _tc_project_body(sums_ref, inv_ref, w_ref, b_ref, out_ref, acc_ref):
    k = pl.program_id(1)
    nk = pl.num_programs(1)

    @pl.when(k == 0)
    def _():
        acc_ref[...] = jnp.zeros_like(acc_ref)

    acc_ref[...] += jnp.dot(sums_ref[0, 0], w_ref[0],
                            preferred_element_type=jnp.float32)

    @pl.when(k == nk - 1)
    def _():
        out_ref[0] = acc_ref[...] * inv_ref[0] + b_ref[...]


def _tc_project(sums, inv, w3, b2):
    return pl.pallas_call(
        _tc_project_body,
        grid=(_B, _G),
        in_specs=[
            pl.BlockSpec((1, 1, _P, _CG), lambda b, k: (b, k, 0, 0)),
            pl.BlockSpec((1, _P, 1), lambda b, k: (b, 0, 0)),
            pl.BlockSpec((1, _CG, _O), lambda b, k: (k, 0, 0)),
            pl.BlockSpec((1, _O), lambda b, k: (0, 0)),
        ],
        out_specs=pl.BlockSpec((1, _P, _O), lambda b, k: (b, 0, 0)),
        out_shape=jax.ShapeDtypeStruct((_B, _P, _O), jnp.float32),
        scratch_shapes=[pltpu.VMEM((_P, _O), jnp.float32)],
    )(sums, inv, w3, b2)


def kernel(byte_hiddens, patch_ids, W_proj, b_proj):
    pid = patch_ids.astype(jnp.int32)
    pid3 = pid.reshape(_B, _T // _CHUNK, _CHUNK)
    zsum = jnp.zeros((_CHUNK, _G, _CG), jnp.float32)
    sums = _sc_pool(byte_hiddens, pid3, zsum)
    inv = _tc_inv(pid.reshape(_B, _G, _TK))
    return _tc_project(sums, inv,
                       W_proj.reshape(_G, _CG, _O), b_proj.reshape(1, _O))


# speedup vs baseline: 1.8164x; 1.0130x over previous
import functools

import jax
import jax.numpy as jnp
from jax import lax
from jax.experimental import pallas as pl
from jax.experimental.pallas import tpu as pltpu
from jax.experimental.pallas import tpu_sc as plsc

_B = 4
_T = 4096
_H = 1024
_P = 512
_O = 768
_NS = 16
_G = 8
_CG = _H // _G
_CHUNK = 16

_TPW = _T // _NS
_NCH = _TPW // _CHUNK
_PW = _P // _NS
_TK = _T // _G


def _sc_pool(h, pid3, zsum):
    mesh = plsc.VectorSubcoreMesh(core_axis_name="c", subcore_axis_name="s")

    @functools.partial(
        pl.kernel,
        out_type=jax.ShapeDtypeStruct((_B, _G, _P, _CG), jnp.float32),
        mesh=mesh,
        scratch_types=[
            pltpu.VMEM((_CHUNK, _G, _CG), jnp.float32),
            pltpu.VMEM((_CHUNK, _G, _CG), jnp.float32),
            pltpu.VMEM((_NCH, _CHUNK), jnp.int32),
            pltpu.VMEM_SHARED((_P, _G, _CG), jnp.float32),
            pltpu.VMEM_SHARED((_P, _G, _CG), jnp.float32),
            pltpu.SemaphoreType.DMA,
            pltpu.SemaphoreType.DMA,
            pltpu.SemaphoreType.DMA,
            pltpu.SemaphoreType.DMA,
        ],
    )
    def k(h_hbm, pid_hbm, zsum_hbm, sums_hbm,
          chunk0_v, chunk1_v, idx_v, acc0, acc1,
          gsem0, gsem1, ssem0, ssem1):
        c = lax.axis_index("c")
        s = lax.axis_index("s")
        tok0 = s * _TPW
        bufs = (chunk0_v, chunk1_v)
        gsems = (gsem0, gsem1)
        ssems = (ssem0, ssem1)

        pltpu.sync_copy(zsum_hbm, chunk0_v)
        for acc in (acc0, acc1):
            for z in range(_PW // _CHUNK):
                pltpu.sync_copy(
                    chunk0_v, acc.at[pl.ds(s * _PW + z * _CHUNK, _CHUNK)])
        plsc.subcore_barrier()

        def gather(b, j, k2):
            t0 = tok0 + j * _CHUNK
            for g in range(_G):
                pltpu.async_copy(
                    h_hbm.at[b, pl.ds(t0, _CHUNK), pl.ds(g * _CG, _CG)],
                    bufs[k2].at[:, g], gsems[k2])

        def drain(sem, buf):
            pltpu.make_async_copy(zsum_hbm, buf, sem).wait()

        for bi, acc in enumerate((acc0, acc1)):
            b = c * 2 + bi
            pltpu.sync_copy(
                pid_hbm.at[b, pl.ds(s * _NCH, _NCH)], idx_v)
            for b2 in range(2):
                gather(b, b2, b2)

            @pl.loop(0, _NCH, step=2)
            def _(jg):
                for b2 in range(2):
                    j = jg + b2
                    drain(gsems[b2], bufs[b2])
                    pltpu.async_copy(bufs[b2], acc.at[idx_v.at[j]],
                                     ssems[b2], add=True)

                    @pl.when(j + 2 < _NCH)
                    def _():
                        drain(ssems[b2], bufs[b2])
                        gather(b, j + 2, b2)

            for b2 in range(2):
                drain(ssems[b2], bufs[b2])
        plsc.subcore_barrier()

        for bi, acc in enumerate((acc0, acc1)):
            b = c * 2 + bi
            for g in range(_G):
                pltpu.sync_copy(
                    acc.at[pl.ds(s * _PW, _PW), g],
                    sums_hbm.at[b, g, pl.ds(s * _PW, _PW)])

    return k(h, pid3, zsum)


def _tc_inv_body(pid_ref, inv_ref):
    cnt = jnp.zeros((_P, 1), jnp.float32)
    patches = lax.broadcasted_iota(jnp.int32, (_P, _TK), 0)
    for r in range(_G):
        ids = pid_ref[0, r]
        m = (patches == ids[None, :]).astype(jnp.float32)
        cnt = cnt + jnp.sum(m, axis=1, keepdims=True)
    inv_ref[0] = 1.0 / jnp.maximum(cnt, 1.0)


def _tc_inv(pid):
    return pl.pallas_call(
        _tc_inv_body,
        grid=(_B,),
        in_specs=[pl.BlockSpec((1, _G, _TK), lambda b: (b, 0, 0))],
        out_specs=pl.BlockSpec((1, _P, 1), lambda b: (b, 0, 0)),
        out_shape=jax.ShapeDtypeStruct((_B, _P, 1), jnp.float32),
    )(pid)


def _tc_project_body(sums_ref, inv_ref, w_ref, b_ref, out_ref, acc_ref):
    k = pl.program_id(1)
    nk = pl.num_programs(1)

    @pl.when(k == 0)
    def _():
        acc_ref[...] = jnp.zeros_like(acc_ref)

    acc_ref[...] += jnp.dot(sums_ref[0, 0].astype(jnp.bfloat16),
                            w_ref[pl.ds(k * _CG, _CG), :],
                            preferred_element_type=jnp.float32)

    @pl.when(k == nk - 1)
    def _():
        out_ref[0] = acc_ref[...] * inv_ref[0] + b_ref[...]


def _tc_project(sums, inv, w_bf, b2):
    return pl.pallas_call(
        _tc_project_body,
        grid=(_B, _G),
        in_specs=[
            pl.BlockSpec((1, 1, _P, _CG), lambda b, k: (b, k, 0, 0)),
            pl.BlockSpec((1, _P, 1), lambda b, k: (b, 0, 0)),
            pl.BlockSpec((_H, _O), lambda b, k: (0, 0)),
            pl.BlockSpec((1, _O), lambda b, k: (0, 0)),
        ],
        out_specs=pl.BlockSpec((1, _P, _O), lambda b, k: (b, 0, 0)),
        out_shape=jax.ShapeDtypeStruct((_B, _P, _O), jnp.float32),
        scratch_shapes=[pltpu.VMEM((_P, _O), jnp.float32)],
    )(sums, inv, w_bf, b2)


def kernel(byte_hiddens, patch_ids, W_proj, b_proj):
    pid = patch_ids.astype(jnp.int32)
    pid3 = pid.reshape(_B, _T // _CHUNK, _CHUNK)
    zsum = jnp.zeros((_CHUNK, _G, _CG), jnp.float32)
    sums = _sc_pool(byte_hiddens, pid3, zsum)
    inv = _tc_inv(pid.reshape(_B, _G, _TK))
    return _tc_project(sums, inv, W_proj.astype(jnp.bfloat16),
                       b_proj.reshape(1, _O))
